# Initial kernel scaffold; baseline (speedup 1.0000x reference)
import jax
import jax.numpy as jnp
from jax.experimental import pallas as pl


def kernel(x, edge_index, W, b):
    raise NotImplementedError("write your pallas kernel here")


# speedup vs baseline: 10.3266x; 10.3266x over previous
import functools

import jax
import jax.numpy as jnp
from jax import lax
from jax.experimental import pallas as pl
from jax.experimental.pallas import tpu as pltpu
from jax.experimental.pallas import tpu_sc as plsc

N = 10000
E = 320000
D = 128

NC = 2
NS = 16
NW = NC * NS

NPAD = 10240
ET = NPAD
EPAD = NW * ET
PADNODE = N
B = 128
NB = ET // B
RPT = NPAD // NS

_MESH = plsc.VectorSubcoreMesh(core_axis_name="c", subcore_axis_name="s")


@functools.partial(
    pl.kernel,
    out_type=jax.ShapeDtypeStruct((NC, NPAD, D), jnp.float32),
    mesh=_MESH,
    scratch_types=[
        pltpu.VMEM((NB, B), jnp.int32),
        pltpu.VMEM((B, D), jnp.float32),
        pltpu.VMEM_SHARED((NPAD, D), jnp.float32),
    ],
)
def _sc_hist(dst_hbm, zeros_hbm, ones_hbm, out_hbm, dst_v, ones_v, hist_sh):
    cid = lax.axis_index("c")
    sid = lax.axis_index("s")
    wid = cid * NS + sid
    pltpu.sync_copy(dst_hbm.at[wid], dst_v)
    pltpu.sync_copy(ones_hbm, ones_v)
    pltpu.sync_copy(zeros_hbm, hist_sh.at[pl.ds(sid * RPT, RPT)])
    plsc.subcore_barrier()

    def body(j, carry):
        pltpu.sync_copy(ones_v, hist_sh.at[dst_v.at[j]], add=True)
        return carry

    lax.fori_loop(0, NB, body, 0)
    plsc.subcore_barrier()
    pltpu.sync_copy(
        hist_sh.at[pl.ds(sid * RPT, RPT)],
        out_hbm.at[cid, pl.ds(sid * RPT, RPT)],
    )


@functools.partial(
    pl.kernel,
    out_type=jax.ShapeDtypeStruct((NC, NPAD, D), jnp.float32),
    mesh=_MESH,
    scratch_types=[
        pltpu.VMEM((NB, B), jnp.int32),
        pltpu.VMEM((NB, B), jnp.int32),
        pltpu.VMEM((B, D), jnp.float32),
        pltpu.VMEM_SHARED((NPAD, D), jnp.float32),
        pltpu.SemaphoreType.DMA,
    ],
)
def _sc_edges(hs_hbm, src_hbm, dst_hbm, zrows_hbm, out_hbm,
              src_v, dst_v, rows_v, acc_sh, sem):
    cid = lax.axis_index("c")
    sid = lax.axis_index("s")
    wid = cid * NS + sid
    pltpu.sync_copy(src_hbm.at[wid], src_v)
    pltpu.sync_copy(dst_hbm.at[wid], dst_v)
    pltpu.sync_copy(zrows_hbm, acc_sh.at[pl.ds(sid * RPT, RPT)])
    plsc.subcore_barrier()

    def body(j, carry):
        pltpu.async_copy(hs_hbm.at[src_v.at[j]], rows_v, sem).wait()
        pltpu.sync_copy(rows_v, acc_sh.at[dst_v.at[j]], add=True)
        return carry

    lax.fori_loop(0, NB, body, 0)
    plsc.subcore_barrier()
    pltpu.sync_copy(
        acc_sh.at[pl.ds(sid * RPT, RPT)],
        out_hbm.at[cid, pl.ds(sid * RPT, RPT)],
    )


def _tc_pre_body(x_ref, w_ref, hg_ref, hs_ref):
    deg = hg_ref[0, :, 0] + hg_ref[1, :, 0] + 1.0
    di = lax.rsqrt(deg)
    h = jnp.dot(x_ref[...], w_ref[...], preferred_element_type=jnp.float32)
    hs_ref[...] = h * di[:, None]


def _tc_pre(x_p, W, hists):
    blk = 512
    return pl.pallas_call(
        _tc_pre_body,
        grid=(NPAD // blk,),
        in_specs=[
            pl.BlockSpec((blk, D), lambda i: (i, 0)),
            pl.BlockSpec((D, D), lambda i: (0, 0)),
            pl.BlockSpec((NC, blk, D), lambda i: (0, i, 0)),
        ],
        out_specs=pl.BlockSpec((blk, D), lambda i: (i, 0)),
        out_shape=jax.ShapeDtypeStruct((NPAD, D), jnp.float32),
    )(x_p, W, hists)


def _tc_post_body(p_ref, hs_ref, hg_ref, b_ref, o_ref):
    deg = hg_ref[0, :, 0] + hg_ref[1, :, 0] + 1.0
    di = lax.rsqrt(deg)
    s = (p_ref[0] + p_ref[1] + hs_ref[...]) * di[:, None] + b_ref[...]
    o_ref[...] = jnp.maximum(s, 0.0)


def _tc_post(parts, hs, hists, b2):
    blk = 512
    return pl.pallas_call(
        _tc_post_body,
        grid=(NPAD // blk,),
        in_specs=[
            pl.BlockSpec((NC, blk, D), lambda i: (0, i, 0)),
            pl.BlockSpec((blk, D), lambda i: (i, 0)),
            pl.BlockSpec((NC, blk, D), lambda i: (0, i, 0)),
            pl.BlockSpec((1, D), lambda i: (0, 0)),
        ],
        out_specs=pl.BlockSpec((blk, D), lambda i: (i, 0)),
        out_shape=jax.ShapeDtypeStruct((NPAD, D), jnp.float32),
    )(parts, hs, hists, b2)


def kernel(x, edge_index, W, b):
    src = edge_index[0].astype(jnp.int32)
    dst = edge_index[1].astype(jnp.int32)
    padi = jnp.full((EPAD - E,), PADNODE, jnp.int32)
    src_p = jnp.concatenate([src, padi]).reshape(NW, NB, B)
    dst_p = jnp.concatenate([dst, padi]).reshape(NW, NB, B)
    x_p = jnp.concatenate(
        [x.astype(jnp.float32), jnp.zeros((NPAD - N, D), jnp.float32)]
    )

    ones_rows = jnp.ones((B, D), jnp.float32)
    zrows = jnp.zeros((RPT, D), jnp.float32)

    hists = _sc_hist(dst_p, zrows, ones_rows)
    hs = _tc_pre(x_p, W.astype(jnp.float32), hists)
    parts = _sc_edges(hs, src_p, dst_p, zrows)
    out = _tc_post(parts, hs, hists, b.reshape(1, D).astype(jnp.float32))
    return out[:N]


# speedup vs baseline: 11.5083x; 1.1144x over previous
import functools

import jax
import jax.numpy as jnp
from jax import lax
from jax.experimental import pallas as pl
from jax.experimental.pallas import tpu as pltpu
from jax.experimental.pallas import tpu_sc as plsc

N = 10000
E = 320000
D = 128

NC = 2
NS = 16
NW = NC * NS

NPAD = 10240
ET = NPAD
EPAD = NW * ET
PADNODE = N
B = 80
U = 4
NB = ET // B
NT = NB // U
RPT = NPAD // NS

_MESH = plsc.VectorSubcoreMesh(core_axis_name="c", subcore_axis_name="s")


def _superstep_indices(sd_hbm, wid, t, idx_v, isem):
    return pltpu.async_copy(sd_hbm.at[wid, t], idx_v, isem)


@functools.partial(
    pl.kernel,
    out_type=jax.ShapeDtypeStruct((NC, NPAD, D), jnp.float32),
    mesh=_MESH,
    scratch_types=[
        pltpu.VMEM((2, U, B), jnp.int32),
        pltpu.VMEM((2, U, B), jnp.int32),
        pltpu.VMEM((B, D), jnp.float32),
        pltpu.VMEM_SHARED((NPAD, D), jnp.float32),
        pltpu.SemaphoreType.DMA((2,)),
        pltpu.SemaphoreType.DMA((U,)),
    ],
)
def _sc_hist(sd_hbm, zeros_hbm, ones_hbm, out_hbm,
             idx0, idx1, ones_v, hist_sh, isems, ssems):
    cid = lax.axis_index("c")
    sid = lax.axis_index("s")
    wid = cid * NS + sid
    pltpu.sync_copy(ones_hbm, ones_v)
    pltpu.sync_copy(zeros_hbm, hist_sh.at[pl.ds(sid * RPT, RPT)])
    _superstep_indices(sd_hbm, wid, 0, idx0, isems.at[0])
    _superstep_indices(sd_hbm, wid, 1, idx1, isems.at[1])
    plsc.subcore_barrier()

    def superstep(t, idx_v, p):
        sd = [
            pltpu.async_copy(
                ones_v, hist_sh.at[idx_v.at[1, k]], ssems.at[k], add=True
            )
            for k in range(U)
        ]
        for d in sd:
            d.wait()
        tn = jnp.minimum(t + 2, NT - 1)
        _superstep_indices(sd_hbm, wid, tn, idx_v, isems.at[p])

    def body(t2, carry):
        ta = 2 * t2
        pltpu.make_async_copy(sd_hbm.at[wid, 0], idx0, isems.at[0]).wait()
        superstep(ta, idx0, 0)
        pltpu.make_async_copy(sd_hbm.at[wid, 0], idx1, isems.at[1]).wait()
        superstep(ta + 1, idx1, 1)
        return carry

    lax.fori_loop(0, NT // 2, body, 0)
    pltpu.make_async_copy(sd_hbm.at[wid, 0], idx0, isems.at[0]).wait()
    pltpu.make_async_copy(sd_hbm.at[wid, 0], idx1, isems.at[1]).wait()
    plsc.subcore_barrier()
    pltpu.sync_copy(
        hist_sh.at[pl.ds(sid * RPT, RPT)],
        out_hbm.at[cid, pl.ds(sid * RPT, RPT)],
    )


@functools.partial(
    pl.kernel,
    out_type=jax.ShapeDtypeStruct((NC, NPAD, D), jnp.float32),
    mesh=_MESH,
    scratch_types=[
        pltpu.VMEM((2, U, B), jnp.int32),
        pltpu.VMEM((2, U, B), jnp.int32),
    ] + [pltpu.VMEM((B, D), jnp.float32) for _ in range(U)] + [
        pltpu.VMEM_SHARED((NPAD, D), jnp.float32),
        pltpu.SemaphoreType.DMA((2,)),
        pltpu.SemaphoreType.DMA((U,)),
        pltpu.SemaphoreType.DMA((U,)),
    ],
)
def _sc_edges(hs_hbm, sd_hbm, zrows_hbm, out_hbm,
              idx0, idx1, r0, r1, r2, r3, acc_sh, isems, gsems, ssems):
    rows = (r0, r1, r2, r3)
    cid = lax.axis_index("c")
    sid = lax.axis_index("s")
    wid = cid * NS + sid
    pltpu.sync_copy(zrows_hbm, acc_sh.at[pl.ds(sid * RPT, RPT)])
    _superstep_indices(sd_hbm, wid, 0, idx0, isems.at[0])
    _superstep_indices(sd_hbm, wid, 1, idx1, isems.at[1])
    plsc.subcore_barrier()

    def superstep(t, idx_v, p):
        gd = [
            pltpu.async_copy(
                hs_hbm.at[idx_v.at[0, k]], rows[k], gsems.at[k]
            )
            for k in range(U)
        ]
        sd = []
        for k in range(U):
            gd[k].wait()
            sd.append(
                pltpu.async_copy(
                    rows[k], acc_sh.at[idx_v.at[1, k]], ssems.at[k], add=True
                )
            )
        for d in sd:
            d.wait()
        tn = jnp.minimum(t + 2, NT - 1)
        _superstep_indices(sd_hbm, wid, tn, idx_v, isems.at[p])

    def body(t2, carry):
        ta = 2 * t2
        pltpu.make_async_copy(sd_hbm.at[wid, 0], idx0, isems.at[0]).wait()
        superstep(ta, idx0, 0)
        pltpu.make_async_copy(sd_hbm.at[wid, 0], idx1, isems.at[1]).wait()
        superstep(ta + 1, idx1, 1)
        return carry

    lax.fori_loop(0, NT // 2, body, 0)
    pltpu.make_async_copy(sd_hbm.at[wid, 0], idx0, isems.at[0]).wait()
    pltpu.make_async_copy(sd_hbm.at[wid, 0], idx1, isems.at[1]).wait()
    plsc.subcore_barrier()
    pltpu.sync_copy(
        acc_sh.at[pl.ds(sid * RPT, RPT)],
        out_hbm.at[cid, pl.ds(sid * RPT, RPT)],
    )


def _tc_pre_body(x_ref, w_ref, hg_ref, hs_ref):
    deg = hg_ref[0, :, 0] + hg_ref[1, :, 0] + 1.0
    di = lax.rsqrt(deg)
    h = jnp.dot(x_ref[...], w_ref[...], preferred_element_type=jnp.float32)
    hs_ref[...] = h * di[:, None]


def _tc_pre(x_p, W, hists):
    blk = 512
    return pl.pallas_call(
        _tc_pre_body,
        grid=(NPAD // blk,),
        in_specs=[
            pl.BlockSpec((blk, D), lambda i: (i, 0)),
            pl.BlockSpec((D, D), lambda i: (0, 0)),
            pl.BlockSpec((NC, blk, D), lambda i: (0, i, 0)),
        ],
        out_specs=pl.BlockSpec((blk, D), lambda i: (i, 0)),
        out_shape=jax.ShapeDtypeStruct((NPAD, D), jnp.float32),
    )(x_p, W, hists)


def _tc_post_body(p_ref, hs_ref, hg_ref, b_ref, o_ref):
    deg = hg_ref[0, :, 0] + hg_ref[1, :, 0] + 1.0
    di = lax.rsqrt(deg)
    s = (p_ref[0] + p_ref[1] + hs_ref[...]) * di[:, None] + b_ref[...]
    o_ref[...] = jnp.maximum(s, 0.0)


def _tc_post(parts, hs, hists, b2):
    blk = 512
    return pl.pallas_call(
        _tc_post_body,
        grid=(NPAD // blk,),
        in_specs=[
            pl.BlockSpec((NC, blk, D), lambda i: (0, i, 0)),
            pl.BlockSpec((blk, D), lambda i: (i, 0)),
            pl.BlockSpec((NC, blk, D), lambda i: (0, i, 0)),
            pl.BlockSpec((1, D), lambda i: (0, 0)),
        ],
        out_specs=pl.BlockSpec((blk, D), lambda i: (i, 0)),
        out_shape=jax.ShapeDtypeStruct((NPAD, D), jnp.float32),
    )(parts, hs, hists, b2)


def kernel(x, edge_index, W, b):
    src = edge_index[0].astype(jnp.int32)
    dst = edge_index[1].astype(jnp.int32)
    padi = jnp.full((EPAD - E,), PADNODE, jnp.int32)
    src_p = jnp.concatenate([src, padi]).reshape(NW, NT, U, B)
    dst_p = jnp.concatenate([dst, padi]).reshape(NW, NT, U, B)
    srcdst = jnp.stack([src_p, dst_p], axis=2)
    x_p = jnp.concatenate(
        [x.astype(jnp.float32), jnp.zeros((NPAD - N, D), jnp.float32)]
    )

    ones_rows = jnp.ones((B, D), jnp.float32)
    zrows = jnp.zeros((RPT, D), jnp.float32)

    hists = _sc_hist(srcdst, zrows, ones_rows)
    hs = _tc_pre(x_p, W.astype(jnp.float32), hists)
    parts = _sc_edges(hs, srcdst, zrows)
    out = _tc_post(parts, hs, hists, b.reshape(1, D).astype(jnp.float32))
    return out[:N]


# speedup vs baseline: 12.0378x; 1.0460x over previous
import functools

import jax
import jax.numpy as jnp
from jax import lax
from jax.experimental import pallas as pl
from jax.experimental.pallas import tpu as pltpu
from jax.experimental.pallas import tpu_sc as plsc

N = 10000
E = 320000
D = 128

NC = 2
NS = 16
NW = NC * NS

NPAD = 10240
ET = NPAD
EPAD = NW * ET
PADNODE = N
B = 80
U = 4
NB = ET // B
NT = NB // U
RPT = NPAD // NS

_MESH = plsc.VectorSubcoreMesh(core_axis_name="c", subcore_axis_name="s")


def _superstep_indices(sd_hbm, wid, t, idx_v, isem):
    return pltpu.async_copy(sd_hbm.at[wid, t], idx_v, isem)


@functools.partial(
    pl.kernel,
    out_type=jax.ShapeDtypeStruct((NC, NPAD, D), jnp.float32),
    mesh=_MESH,
    scratch_types=[
        pltpu.VMEM((2, U, B), jnp.int32),
        pltpu.VMEM((2, U, B), jnp.int32),
        pltpu.VMEM((B, D), jnp.float32),
        pltpu.VMEM_SHARED((NPAD, D), jnp.float32),
        pltpu.SemaphoreType.DMA((2,)),
        pltpu.SemaphoreType.DMA((U,)),
    ],
)
def _sc_hist(sd_hbm, zeros_hbm, ones_hbm, out_hbm,
             idx0, idx1, ones_v, hist_sh, isems, ssems):
    cid = lax.axis_index("c")
    sid = lax.axis_index("s")
    wid = cid * NS + sid
    pltpu.sync_copy(ones_hbm, ones_v)
    pltpu.sync_copy(zeros_hbm, hist_sh.at[pl.ds(sid * RPT, RPT)])
    _superstep_indices(sd_hbm, wid, 0, idx0, isems.at[0])
    plsc.subcore_barrier()

    def idx_wait(idx_v, p):
        pltpu.make_async_copy(sd_hbm.at[wid, 0], idx_v, isems.at[p]).wait()

    def add_wait(k):
        pltpu.make_async_copy(ones_v, hist_sh.at[pl.ds(0, B)],
                              ssems.at[k]).wait()

    def superstep(t, bufs, first=False, last=False):
        idx_v, nxt_v = bufs
        p = t % 2
        idx_wait(idx_v, p)
        if not first:
            for k in range(U):
                add_wait(k)
        if not last:
            _superstep_indices(sd_hbm, wid, t + 1, nxt_v, isems.at[1 - p])
        for k in range(U):
            pltpu.async_copy(
                ones_v, hist_sh.at[idx_v.at[1, k]], ssems.at[k], add=True
            )

    superstep(0, (idx0, idx1), first=True)

    def body(t2, carry):
        ta = 2 * t2 + 1
        superstep(ta, (idx1, idx0))
        superstep(ta + 1, (idx0, idx1))
        return carry

    lax.fori_loop(0, (NT - 2) // 2, body, 0)
    superstep(NT - 1, (idx1, idx0), last=True)
    for k in range(U):
        add_wait(k)
    plsc.subcore_barrier()
    pltpu.sync_copy(
        hist_sh.at[pl.ds(sid * RPT, RPT)],
        out_hbm.at[cid, pl.ds(sid * RPT, RPT)],
    )


@functools.partial(
    pl.kernel,
    out_type=jax.ShapeDtypeStruct((NC, NPAD, D), jnp.float32),
    mesh=_MESH,
    scratch_types=[
        pltpu.VMEM((2, U, B), jnp.int32),
        pltpu.VMEM((2, U, B), jnp.int32),
    ] + [pltpu.VMEM((B, D), jnp.float32) for _ in range(U)] + [
        pltpu.VMEM_SHARED((NPAD, D), jnp.float32),
        pltpu.SemaphoreType.DMA((2,)),
        pltpu.SemaphoreType.DMA((U,)),
        pltpu.SemaphoreType.DMA((U,)),
    ],
)
def _sc_edges(hs_hbm, sd_hbm, zrows_hbm, out_hbm,
              idx0, idx1, r0, r1, r2, r3, acc_sh, isems, gsems, ssems):
    rows = (r0, r1, r2, r3)
    cid = lax.axis_index("c")
    sid = lax.axis_index("s")
    wid = cid * NS + sid
    pltpu.sync_copy(zrows_hbm, acc_sh.at[pl.ds(sid * RPT, RPT)])
    _superstep_indices(sd_hbm, wid, 0, idx0, isems.at[0])
    plsc.subcore_barrier()

    def idx_wait(idx_v, p):
        pltpu.make_async_copy(sd_hbm.at[wid, 0], idx_v, isems.at[p]).wait()

    def scat_wait(k):
        pltpu.make_async_copy(rows[k], acc_sh.at[pl.ds(0, B)],
                              ssems.at[k]).wait()

    def superstep(t, bufs, first=False, last=False):
        idx_v, nxt_v = bufs
        p = t % 2
        idx_wait(idx_v, p)
        gd = []
        for k in range(U):
            if not first:
                scat_wait(k)
            gd.append(
                pltpu.async_copy(
                    hs_hbm.at[idx_v.at[0, k]], rows[k], gsems.at[k]
                )
            )
        if not last:
            _superstep_indices(sd_hbm, wid, t + 1, nxt_v, isems.at[1 - p])
        for k in range(U):
            gd[k].wait()
            pltpu.async_copy(
                rows[k], acc_sh.at[idx_v.at[1, k]], ssems.at[k], add=True
            )

    superstep(0, (idx0, idx1), first=True)

    def body(t2, carry):
        ta = 2 * t2 + 1
        superstep(ta, (idx1, idx0))
        superstep(ta + 1, (idx0, idx1))
        return carry

    lax.fori_loop(0, (NT - 2) // 2, body, 0)
    superstep(NT - 1, (idx1, idx0), last=True)
    for k in range(U):
        scat_wait(k)
    plsc.subcore_barrier()
    pltpu.sync_copy(
        acc_sh.at[pl.ds(sid * RPT, RPT)],
        out_hbm.at[cid, pl.ds(sid * RPT, RPT)],
    )


def _tc_pre_body(x_ref, w_ref, hg_ref, hs_ref):
    deg = hg_ref[0, :, 0] + hg_ref[1, :, 0] + 1.0
    di = lax.rsqrt(deg)
    h = jnp.dot(x_ref[...], w_ref[...], preferred_element_type=jnp.float32)
    hs_ref[...] = h * di[:, None]


def _tc_pre(x_p, W, hists):
    blk = 512
    return pl.pallas_call(
        _tc_pre_body,
        grid=(NPAD // blk,),
        in_specs=[
            pl.BlockSpec((blk, D), lambda i: (i, 0)),
            pl.BlockSpec((D, D), lambda i: (0, 0)),
            pl.BlockSpec((NC, blk, D), lambda i: (0, i, 0)),
        ],
        out_specs=pl.BlockSpec((blk, D), lambda i: (i, 0)),
        out_shape=jax.ShapeDtypeStruct((NPAD, D), jnp.float32),
    )(x_p, W, hists)


def _tc_post_body(p_ref, hs_ref, hg_ref, b_ref, o_ref):
    deg = hg_ref[0, :, 0] + hg_ref[1, :, 0] + 1.0
    di = lax.rsqrt(deg)
    s = (p_ref[0] + p_ref[1] + hs_ref[...]) * di[:, None] + b_ref[...]
    o_ref[...] = jnp.maximum(s, 0.0)


def _tc_post(parts, hs, hists, b2):
    blk = 512
    return pl.pallas_call(
        _tc_post_body,
        grid=(NPAD // blk,),
        in_specs=[
            pl.BlockSpec((NC, blk, D), lambda i: (0, i, 0)),
            pl.BlockSpec((blk, D), lambda i: (i, 0)),
            pl.BlockSpec((NC, blk, D), lambda i: (0, i, 0)),
            pl.BlockSpec((1, D), lambda i: (0, 0)),
        ],
        out_specs=pl.BlockSpec((blk, D), lambda i: (i, 0)),
        out_shape=jax.ShapeDtypeStruct((NPAD, D), jnp.float32),
    )(parts, hs, hists, b2)


def kernel(x, edge_index, W, b):
    src = edge_index[0].astype(jnp.int32)
    dst = edge_index[1].astype(jnp.int32)
    padi = jnp.full((EPAD - E,), PADNODE, jnp.int32)
    src_p = jnp.concatenate([src, padi]).reshape(NW, NT, U, B)
    dst_p = jnp.concatenate([dst, padi]).reshape(NW, NT, U, B)
    srcdst = jnp.stack([src_p, dst_p], axis=2)
    x_p = jnp.concatenate(
        [x.astype(jnp.float32), jnp.zeros((NPAD - N, D), jnp.float32)]
    )

    ones_rows = jnp.ones((B, D), jnp.float32)
    zrows = jnp.zeros((RPT, D), jnp.float32)

    hists = _sc_hist(srcdst, zrows, ones_rows)
    hs = _tc_pre(x_p, W.astype(jnp.float32), hists)
    parts = _sc_edges(hs, srcdst, zrows)
    out = _tc_post(parts, hs, hists, b.reshape(1, D).astype(jnp.float32))
    return out[:N]


# speedup vs baseline: 13.4727x; 1.1192x over previous
import functools

import jax
import jax.numpy as jnp
from jax import lax
from jax.experimental import pallas as pl
from jax.experimental.pallas import tpu as pltpu
from jax.experimental.pallas import tpu_sc as plsc

N = 10000
E = 320000
D = 128

NC = 2
NS = 16
NW = NC * NS

NPAD = 10240
EPAD = NW * NPAD
PADNODE = N
B = 80
U = 4
NTT = EPAD // (U * B)
NTH = NTT // NW
NT0 = 12
NT1 = NTT // NS - NT0
RPT = NPAD // NS

_MESH = plsc.VectorSubcoreMesh(core_axis_name="c", subcore_axis_name="s")


def _superstep_indices(sd_hbm, g, idx_v, isem):
    return pltpu.async_copy(sd_hbm.at[g], idx_v, isem)


@functools.partial(
    pl.kernel,
    out_type=jax.ShapeDtypeStruct((NC, NPAD, D), jnp.float32),
    mesh=_MESH,
    scratch_types=[
        pltpu.VMEM((2, U, B), jnp.int32),
        pltpu.VMEM((2, U, B), jnp.int32),
        pltpu.VMEM((B, D), jnp.float32),
        pltpu.VMEM_SHARED((NPAD, D), jnp.float32),
        pltpu.SemaphoreType.DMA((2,)),
        pltpu.SemaphoreType.DMA((U,)),
    ],
)
def _sc_hist(sd_hbm, zeros_hbm, ones_hbm, out_hbm,
             idx0, idx1, ones_v, hist_sh, isems, ssems):
    cid = lax.axis_index("c")
    sid = lax.axis_index("s")
    wid = cid * NS + sid
    base = wid * NTH
    pltpu.sync_copy(ones_hbm, ones_v)
    pltpu.sync_copy(zeros_hbm, hist_sh.at[pl.ds(sid * RPT, RPT)])
    _superstep_indices(sd_hbm, base, idx0, isems.at[0])
    plsc.subcore_barrier()

    def idx_wait(idx_v, p):
        pltpu.make_async_copy(sd_hbm.at[0], idx_v, isems.at[p]).wait()

    def add_wait(k):
        pltpu.make_async_copy(ones_v, hist_sh.at[pl.ds(0, B)],
                              ssems.at[k]).wait()

    def superstep(t, bufs, first=False, last=False):
        idx_v, nxt_v = bufs
        p = t % 2
        idx_wait(idx_v, p)
        if not first:
            for k in range(U):
                add_wait(k)
        if not last:
            _superstep_indices(sd_hbm, base + t + 1, nxt_v, isems.at[1 - p])
        for k in range(U):
            pltpu.async_copy(
                ones_v, hist_sh.at[idx_v.at[1, k]], ssems.at[k], add=True
            )

    superstep(0, (idx0, idx1), first=True)

    def body(t2, carry):
        ta = 2 * t2 + 1
        superstep(ta, (idx1, idx0))
        superstep(ta + 1, (idx0, idx1))
        return carry

    lax.fori_loop(0, (NTH - 2) // 2, body, 0)
    superstep(NTH - 1, (idx1, idx0), last=True)
    for k in range(U):
        add_wait(k)
    plsc.subcore_barrier()
    pltpu.sync_copy(
        hist_sh.at[pl.ds(sid * RPT, RPT)],
        out_hbm.at[cid, pl.ds(sid * RPT, RPT)],
    )


@functools.partial(
    pl.kernel,
    out_type=jax.ShapeDtypeStruct((NC, NPAD, D), jnp.float32),
    mesh=_MESH,
    scratch_types=[
        pltpu.VMEM((2, U, B), jnp.int32),
        pltpu.VMEM((2, U, B), jnp.int32),
    ] + [pltpu.VMEM((B, D), jnp.float32) for _ in range(U)] + [
        pltpu.VMEM_SHARED((NPAD, D), jnp.float32),
        pltpu.SemaphoreType.DMA((2,)),
        pltpu.SemaphoreType.DMA((U,)),
        pltpu.SemaphoreType.DMA((U,)),
    ],
)
def _sc_edges(hs_hbm, sd_hbm, zrows_hbm, out_hbm,
              idx0, idx1, r0, r1, r2, r3, acc_sh, isems, gsems, ssems):
    rows = (r0, r1, r2, r3)
    cid = lax.axis_index("c")
    sid = lax.axis_index("s")
    wid = cid * NS + sid
    nt = jnp.where(cid == 0, NT0, NT1)
    base = jnp.where(cid == 0, sid * NT0, NS * NT0 + sid * NT1)
    pltpu.sync_copy(zrows_hbm, acc_sh.at[pl.ds(sid * RPT, RPT)])
    _superstep_indices(sd_hbm, base, idx0, isems.at[0])
    plsc.subcore_barrier()

    def idx_wait(idx_v, p):
        pltpu.make_async_copy(sd_hbm.at[0], idx_v, isems.at[p]).wait()

    def scat_wait(k):
        pltpu.make_async_copy(rows[k], acc_sh.at[pl.ds(0, B)],
                              ssems.at[k]).wait()

    def superstep(t, bufs, first=False, last=False):
        idx_v, nxt_v = bufs
        p = t % 2
        idx_wait(idx_v, p)
        gd = []
        for k in range(U):
            if not first:
                scat_wait(k)
            gd.append(
                pltpu.async_copy(
                    hs_hbm.at[idx_v.at[0, k]], rows[k], gsems.at[k]
                )
            )
        if not last:
            _superstep_indices(sd_hbm, base + t + 1, nxt_v, isems.at[1 - p])
        for k in range(U):
            gd[k].wait()
            pltpu.async_copy(
                rows[k], acc_sh.at[idx_v.at[1, k]], ssems.at[k], add=True
            )

    superstep(0, (idx0, idx1), first=True)

    def body(t2, carry):
        ta = 2 * t2 + 1
        superstep(ta, (idx1, idx0))
        superstep(ta + 1, (idx0, idx1))
        return carry

    lax.fori_loop(0, (nt - 2) // 2, body, 0)
    superstep(nt - 1, (idx1, idx0), last=True)
    for k in range(U):
        scat_wait(k)
    plsc.subcore_barrier()
    pltpu.sync_copy(
        acc_sh.at[pl.ds(sid * RPT, RPT)],
        out_hbm.at[cid, pl.ds(sid * RPT, RPT)],
    )


def _tc_pre_body(x_ref, w_ref, hg_ref, hs_ref):
    deg = hg_ref[0, :, 0] + hg_ref[1, :, 0] + 1.0
    di = lax.rsqrt(deg)
    h = jnp.dot(x_ref[...], w_ref[...], preferred_element_type=jnp.float32)
    hs_ref[...] = h * di[:, None]


def _tc_pre(x_p, W, hists):
    blk = 512
    return pl.pallas_call(
        _tc_pre_body,
        grid=(NPAD // blk,),
        in_specs=[
            pl.BlockSpec((blk, D), lambda i: (i, 0)),
            pl.BlockSpec((D, D), lambda i: (0, 0)),
            pl.BlockSpec((NC, blk, D), lambda i: (0, i, 0)),
        ],
        out_specs=pl.BlockSpec((blk, D), lambda i: (i, 0)),
        out_shape=jax.ShapeDtypeStruct((NPAD, D), jnp.float32),
    )(x_p, W, hists)


def _tc_post_body(p_ref, hs_ref, hg_ref, b_ref, o_ref):
    deg = hg_ref[0, :, 0] + hg_ref[1, :, 0] + 1.0
    di = lax.rsqrt(deg)
    s = (p_ref[0] + p_ref[1] + hs_ref[...]) * di[:, None] + b_ref[...]
    o_ref[...] = jnp.maximum(s, 0.0)


def _tc_post(parts, hs, hists, b2):
    blk = 512
    return pl.pallas_call(
        _tc_post_body,
        grid=(NPAD // blk,),
        in_specs=[
            pl.BlockSpec((NC, blk, D), lambda i: (0, i, 0)),
            pl.BlockSpec((blk, D), lambda i: (i, 0)),
            pl.BlockSpec((NC, blk, D), lambda i: (0, i, 0)),
            pl.BlockSpec((1, D), lambda i: (0, 0)),
        ],
        out_specs=pl.BlockSpec((blk, D), lambda i: (i, 0)),
        out_shape=jax.ShapeDtypeStruct((NPAD, D), jnp.float32),
    )(parts, hs, hists, b2)


def kernel(x, edge_index, W, b):
    src = edge_index[0].astype(jnp.int32)
    dst = edge_index[1].astype(jnp.int32)
    padi = jnp.full((EPAD - E,), PADNODE, jnp.int32)
    src_p = jnp.concatenate([src, padi]).reshape(NTT, U, B)
    dst_p = jnp.concatenate([dst, padi]).reshape(NTT, U, B)
    srcdst = jnp.stack([src_p, dst_p], axis=1)
    x_p = jnp.concatenate(
        [x.astype(jnp.float32), jnp.zeros((NPAD - N, D), jnp.float32)]
    )

    ones_rows = jnp.ones((B, D), jnp.float32)
    zrows = jnp.zeros((RPT, D), jnp.float32)

    hists = _sc_hist(srcdst, zrows, ones_rows)
    hs = _tc_pre(x_p, W.astype(jnp.float32), hists)
    parts = _sc_edges(hs, srcdst, zrows)
    out = _tc_post(parts, hs, hists, b.reshape(1, D).astype(jnp.float32))
    return out[:N]
